# baseline (speedup 1.0000x reference)
import jax
import jax.numpy as jnp
from jax import lax
from jax.experimental import pallas as pl
from jax.experimental.pallas import tpu as pltpu
from jax.experimental.pallas import tpu_sc as plsc

CONF_THRES = 0.25
IOU_THRES = 0.45
MAX_DET = 300
N_BOXES = 20000
PAD_N = 20480
NSUB = 16
CHUNK = PAD_N // NSUB
L = 16
STEPS = CHUNK // L
LIST_CAP = CHUNK + 4 * L


def _nms_sc(cx_hbm, cy_hbm, w_hbm, h_hbm, s_hbm, out_hbm,
            x1v, y1v, x2v, y2v, actc, idxc, pubv, mrgv, outv, shared):
    cid = lax.axis_index("c")
    sid = lax.axis_index("s")
    base = sid * CHUNK
    first = jnp.logical_and(cid == 0, sid == 0)

    pltpu.sync_copy(cx_hbm, x1v)
    pltpu.sync_copy(cy_hbm, y1v)
    pltpu.sync_copy(w_hbm, x2v)
    pltpu.sync_copy(h_hbm, y2v)
    pltpu.sync_copy(s_hbm.at[pl.ds(base, CHUNK)], actc.at[pl.ds(0, CHUNK)])

    iota = lax.broadcasted_iota(jnp.int32, (L,), 0)

    def _perm(x, idx):
        return x.at[idx].get(mode="promise_in_bounds")

    def _xmax(x):
        for sh in (8, 4, 2, 1):
            x = jnp.maximum(x, _perm(x, iota ^ sh))
        return x

    def _xmin(x):
        for sh in (8, 4, 2, 1):
            x = jnp.minimum(x, _perm(x, iota ^ sh))
        return x

    @plsc.parallel_loop(0, PAD_N // L, unroll=4)
    def _init_xyxy(k):
        sl = pl.ds(k * L, L)
        cx = x1v[sl] * 640.0
        cy = y1v[sl] * 640.0
        w = x2v[sl] * 100.0 + 2.0
        h = y2v[sl] * 100.0 + 2.0
        x1v[sl] = cx - w * 0.5
        y1v[sl] = cy - h * 0.5
        x2v[sl] = cx + w * 0.5
        y2v[sl] = cy + h * 0.5

    def _pad_tail(pos):
        for t in range(4):
            dsl = pl.ds(pos + t * L, L)
            actc[dsl] = jnp.full((L,), -1.0, jnp.float32)
            idxc[dsl] = jnp.full((L,), PAD_N - 1, jnp.int32)
        return (pos + 63) // 64 * 4

    def bbody(k, pos):
        sl = pl.ds(k * L, L)
        s = actc[sl]
        keep = s > CONF_THRES
        plsc.store_compressed(actc.at[pl.ds(pos, L)], s, mask=keep)
        plsc.store_compressed(idxc.at[pl.ds(pos, L)], base + k * L + iota,
                              mask=keep)
        cnt = plsc.all_reduce_population_count(keep)
        return pos + cnt[0]
    pos0 = lax.fori_loop(0, STEPS, bbody, jnp.int32(0))
    nsteps0 = _pad_tail(pos0)

    bv0 = jnp.full((L,), -3e38, jnp.float32)
    bg0 = jnp.full((L,), 2**30, jnp.int32)

    def fused_sweep(nsteps, v, j, bx1, by1, bx2, by2, a1):
        @plsc.parallel_loop(0, nsteps, step=4,
                            carry=((bv0, bg0),) * 4, unroll=1)
        def chains(k0, am):
            out = []
            for c in range(4):
                bv2, bg2 = am[c]
                sl = pl.ds((k0 + c) * L, L)
                gi = idxc[sl]
                act = actc[sl]
                x1 = plsc.load_gather(x1v, [gi])
                y1 = plsc.load_gather(y1v, [gi])
                x2 = plsc.load_gather(x2v, [gi])
                y2 = plsc.load_gather(y2v, [gi])
                xx1 = jnp.maximum(bx1, x1)
                yy1 = jnp.maximum(by1, y1)
                xx2 = jnp.minimum(bx2, x2)
                yy2 = jnp.minimum(by2, y2)
                inter = (jnp.maximum(xx2 - xx1, 0.0)
                         * jnp.maximum(yy2 - yy1, 0.0))
                a2 = (x2 - x1) * (y2 - y1)
                iou = inter / (a1 + a2 - inter + 1e-7)
                sup = jnp.logical_and(
                    jnp.logical_or(iou > IOU_THRES, gi == j), v)
                nact = jnp.where(sup, -1.0, act)
                actc[sl] = nact
                upd = nact > bv2
                out.append((jnp.where(upd, nact, bv2),
                            jnp.where(upd, gi, bg2)))
            return tuple(out)

        def comb(p, q):
            bvp, bgp = p
            bvq, bgq = q
            upd = (bvq > bvp) | ((bvq == bvp) & (bgq < bgp))
            return (jnp.where(upd, bvq, bvp), jnp.where(upd, bgq, bgp))
        (p0, p1, p2, p3) = chains
        return comb(comb(p0, p1), comb(p2, p3))

    vfalse = iota < 0
    j0 = jnp.zeros((L,), jnp.int32)
    c0 = plsc.load_gather(x1v, [j0])
    am_init = fused_sweep(nsteps0, vfalse, j0, c0, c0, c0, c0, c0)

    def compact(nst):
        def cbody(k, pos):
            sl = pl.ds(k * L, L)
            a = actc[sl]
            gi = idxc[sl]
            keep = a > 0.0
            plsc.store_compressed(actc.at[pl.ds(pos, L)], a, mask=keep)
            plsc.store_compressed(idxc.at[pl.ds(pos, L)], gi, mask=keep)
            cnt = plsc.all_reduce_population_count(keep)
            return pos + cnt[0]
        pos = lax.fori_loop(0, nst, cbody, jnp.int32(0))
        return _pad_tail(pos)

    def round_body(i, carry):
        (bv, bg), nsteps = carry
        m_loc = _xmax(bv)
        j_loc = _xmin(jnp.where(bv == m_loc, bg, jnp.int32(2**30)))

        pubv[:] = jnp.where(iota == 0, m_loc,
                  jnp.where(iota == 1, j_loc.astype(jnp.float32), 0.0))
        par = (i & 1) * (NSUB * L)
        pltpu.sync_copy(pubv, shared.at[pl.ds(par + sid * L, L)])
        plsc.subcore_barrier()
        pltpu.sync_copy(shared.at[pl.ds(par, NSUB * L)], mrgv)
        vals = plsc.load_gather(mrgv, [iota * L])
        idxs = plsc.load_gather(mrgv, [iota * L + 1])
        best_m = _xmax(vals)
        j = _xmin(jnp.where(vals == best_m, idxs, 3e38)).astype(jnp.int32)
        j = jnp.minimum(j, jnp.int32(PAD_N - 1))
        v = best_m > 0.0

        bx1 = plsc.load_gather(x1v, [j])
        by1 = plsc.load_gather(y1v, [j])
        bx2 = plsc.load_gather(x2v, [j])
        by2 = plsc.load_gather(y2v, [j])
        a1 = (bx2 - bx1) * (by2 - by1)

        am_next = fused_sweep(nsteps, v, j, bx1, by1, bx2, by2, a1)

        nsteps = lax.cond(i % 16 == 15, compact, lambda n: n, nsteps)

        @pl.when(first)
        def _():
            vf = jnp.where(v, 1.0, 0.0)
            row = jnp.where(iota == 0, bx1 * vf,
                  jnp.where(iota == 1, by1 * vf,
                  jnp.where(iota == 2, bx2 * vf,
                  jnp.where(iota == 3, by2 * vf,
                  jnp.where(iota == 4, best_m * vf, 0.0)))))
            plsc.store_scatter(outv, [i * L + iota], row)
        return (am_next, nsteps)

    lax.fori_loop(0, MAX_DET, round_body, (am_init, nsteps0))

    @pl.when(first)
    def _():
        pltpu.sync_copy(outv, out_hbm)


@jax.jit
def kernel(boxes, scores):
    pad = PAD_N - N_BOXES
    bp = jnp.pad(boxes, ((0, pad), (0, 0)))
    sp = jnp.pad(scores, (0, pad))
    f32 = jnp.float32
    mesh = plsc.VectorSubcoreMesh(core_axis_name="c", subcore_axis_name="s")
    k = pl.kernel(
        _nms_sc,
        mesh=mesh,
        compiler_params=pltpu.CompilerParams(needs_layout_passes=False),
        out_type=jax.ShapeDtypeStruct((MAX_DET * L,), f32),
        scratch_types=[
            pltpu.VMEM((PAD_N,), f32),
            pltpu.VMEM((PAD_N,), f32),
            pltpu.VMEM((PAD_N,), f32),
            pltpu.VMEM((PAD_N,), f32),
            pltpu.VMEM((LIST_CAP,), f32),
            pltpu.VMEM((LIST_CAP,), jnp.int32),
            pltpu.VMEM((L,), f32),
            pltpu.VMEM((NSUB * L,), f32),
            pltpu.VMEM((MAX_DET * L,), f32),
            pltpu.VMEM_SHARED((2 * NSUB * L,), f32),
        ],
    )
    out = k(bp[:, 0], bp[:, 1], bp[:, 2], bp[:, 3], sp)
    return out.reshape(MAX_DET, L)[:, :5]
